# baseline (device time: 65166 ns/iter reference)
import jax
import jax.numpy as jnp
from jax import lax
from jax.experimental import pallas as pl
from jax.experimental.pallas import tpu as pltpu

N_DEV = 16
BK = 512


def kernel(x, w_mat, scale_x, scale_w):
    k_full, k_shard = x.shape
    _, n = w_mat.shape
    m_per = k_full // N_DEV
    n_steps = k_full // BK
    n_halves = 2
    f8 = jnp.float8_e4m3fn

    def body(x_ref, w_ref, sx_ref, sw_ref, out_ref, x8_ref, xt_ref,
             send_sems, recv_sems):
        my = lax.axis_index("i")
        jn = pl.program_id(0)
        pid = pl.program_id(1)

        @pl.when((pid == 0) & (jn == 0))
        def _a2a_start():
            x8_ref[...] = x_ref[...].astype(f8)

            xt_ref[:, pl.ds(my * k_shard, k_shard)] = (
                x8_ref[pl.ds(my * m_per, m_per), :])

            for d in range(N_DEV):
                @pl.when(my != d)
                def _():
                    rdma = pltpu.make_async_remote_copy(
                        src_ref=x8_ref.at[pl.ds(d * m_per, m_per), :],
                        dst_ref=xt_ref.at[:, pl.ds(my * k_shard, k_shard)],
                        send_sem=send_sems.at[d],
                        recv_sem=recv_sems.at[my],
                        device_id=(d,),
                        device_id_type=pl.DeviceIdType.MESH,
                    )
                    rdma.start()

        for t in range(BK // k_shard):
            s = pid * (BK // k_shard) + t
            @pl.when((s != my) & (jn == 0))
            def _wait_chunk():
                recv = pltpu.make_async_remote_copy(
                    src_ref=x8_ref.at[pl.ds(0, m_per), :],
                    dst_ref=xt_ref.at[:, pl.ds(s * k_shard, k_shard)],
                    send_sem=send_sems.at[0],
                    recv_sem=recv_sems.at[s],
                    device_id=(0,),
                    device_id_type=pl.DeviceIdType.MESH,
                )
                recv.wait_recv()

        w8 = w_ref[...].astype(f8)
        part = jnp.dot(xt_ref[:, pl.ds(pid * BK, BK)], w8,
                       preferred_element_type=jnp.float32)

        @pl.when(pid == 0)
        def _():
            out_ref[...] = part

        @pl.when(pid != 0)
        def _():
            out_ref[...] += part

        @pl.when(pid == n_steps - 1)
        def _dequant():
            out_ref[...] *= sx_ref[0, 0] * sw_ref[0, 0]

        @pl.when((pid == n_steps - 1) & (jn == n_halves - 1))
        def _finish():
            for d in range(N_DEV):
                @pl.when(my != d)
                def _():
                    send = pltpu.make_async_remote_copy(
                        src_ref=x8_ref.at[pl.ds(d * m_per, m_per), :],
                        dst_ref=xt_ref.at[:, pl.ds(my * k_shard, k_shard)],
                        send_sem=send_sems.at[d],
                        recv_sem=recv_sems.at[d],
                        device_id=(d,),
                        device_id_type=pl.DeviceIdType.MESH,
                    )
                    send.wait_send()

    out = pl.pallas_call(
        body,
        grid=(n_halves, n_steps),
        out_shape=jax.ShapeDtypeStruct((m_per, n), jnp.float32),
        in_specs=[
            pl.BlockSpec((k_full, k_shard), lambda j, k: (0, 0),
                         memory_space=pltpu.VMEM),
            pl.BlockSpec((BK, n // n_halves), lambda j, k: (k, j),
                         memory_space=pltpu.VMEM),
            pl.BlockSpec(memory_space=pltpu.SMEM),
            pl.BlockSpec(memory_space=pltpu.SMEM),
        ],
        out_specs=pl.BlockSpec((m_per, n // n_halves), lambda j, k: (0, j),
                               memory_space=pltpu.VMEM),
        scratch_shapes=[
            pltpu.VMEM((k_full, k_shard), f8),
            pltpu.VMEM((m_per, k_full), f8),
            pltpu.SemaphoreType.DMA((N_DEV,)),
            pltpu.SemaphoreType.DMA((N_DEV,)),
        ],
        compiler_params=pltpu.CompilerParams(
            vmem_limit_bytes=60 * 1024 * 1024,
        ),
    )(x, w_mat, scale_x.reshape(1, 1), scale_w.reshape(1, 1))
    return out


# device time: 64515 ns/iter; 1.0101x vs baseline; 1.0101x over previous
import jax
import jax.numpy as jnp
from jax import lax
from jax.experimental import pallas as pl
from jax.experimental.pallas import tpu as pltpu

N_DEV = 16
BK = 512


def kernel(x, w_mat, scale_x, scale_w):
    k_full, k_shard = x.shape
    _, n = w_mat.shape
    m_per = k_full // N_DEV
    n_steps = k_full // BK
    f8 = jnp.float8_e4m3fn

    def body(x_ref, w_ref, sx_ref, sw_ref, out_ref, x8_ref, xt_ref,
             send_sems, recv_sems):
        my = lax.axis_index("i")
        pid = pl.program_id(0)

        @pl.when(pid == 0)
        def _a2a_start():
            x8_ref[...] = x_ref[...].astype(f8)

            xt_ref[:, pl.ds(my * k_shard, k_shard)] = (
                x8_ref[pl.ds(my * m_per, m_per), :])

            for d in range(N_DEV):
                @pl.when(my != d)
                def _():
                    rdma = pltpu.make_async_remote_copy(
                        src_ref=x8_ref.at[pl.ds(d * m_per, m_per), :],
                        dst_ref=xt_ref.at[:, pl.ds(my * k_shard, k_shard)],
                        send_sem=send_sems.at[d],
                        recv_sem=recv_sems.at[my],
                        device_id=(d,),
                        device_id_type=pl.DeviceIdType.MESH,
                    )
                    rdma.start()

        for t in range(BK // k_shard):
            s = pid * (BK // k_shard) + t
            @pl.when(s != my)
            def _wait_chunk():
                recv = pltpu.make_async_remote_copy(
                    src_ref=x8_ref.at[pl.ds(0, m_per), :],
                    dst_ref=xt_ref.at[:, pl.ds(s * k_shard, k_shard)],
                    send_sem=send_sems.at[0],
                    recv_sem=recv_sems.at[s],
                    device_id=(0,),
                    device_id_type=pl.DeviceIdType.MESH,
                )
                recv.wait_recv()

        w8 = w_ref[...].astype(f8)
        part = jnp.dot(xt_ref[:, pl.ds(pid * BK, BK)], w8,
                       preferred_element_type=jnp.float32)

        @pl.when(pid == 0)
        def _():
            out_ref[...] = part

        @pl.when(pid != 0)
        def _():
            out_ref[...] += part

        @pl.when(pid == n_steps - 1)
        def _finish():
            out_ref[...] *= sx_ref[0, 0] * sw_ref[0, 0]
            for d in range(N_DEV):
                @pl.when(my != d)
                def _():
                    send = pltpu.make_async_remote_copy(
                        src_ref=x8_ref.at[pl.ds(d * m_per, m_per), :],
                        dst_ref=xt_ref.at[:, pl.ds(my * k_shard, k_shard)],
                        send_sem=send_sems.at[d],
                        recv_sem=recv_sems.at[d],
                        device_id=(d,),
                        device_id_type=pl.DeviceIdType.MESH,
                    )
                    send.wait_send()

    out = pl.pallas_call(
        body,
        grid=(n_steps,),
        out_shape=jax.ShapeDtypeStruct((m_per, n), jnp.float32),
        in_specs=[
            pl.BlockSpec((k_full, k_shard), lambda k: (0, 0),
                         memory_space=pltpu.VMEM),
            pl.BlockSpec((BK, n), lambda k: (k, 0),
                         memory_space=pltpu.VMEM),
            pl.BlockSpec(memory_space=pltpu.SMEM),
            pl.BlockSpec(memory_space=pltpu.SMEM),
        ],
        out_specs=pl.BlockSpec((m_per, n), lambda k: (0, 0),
                               memory_space=pltpu.VMEM),
        scratch_shapes=[
            pltpu.VMEM((k_full, k_shard), f8),
            pltpu.VMEM((m_per, k_full), f8),
            pltpu.SemaphoreType.DMA((N_DEV,)),
            pltpu.SemaphoreType.DMA((N_DEV,)),
        ],
        compiler_params=pltpu.CompilerParams(
            vmem_limit_bytes=60 * 1024 * 1024,
        ),
    )(x, w_mat, scale_x.reshape(1, 1), scale_w.reshape(1, 1))
    return out
